# Initial kernel scaffold; baseline (speedup 1.0000x reference)
import jax
import jax.numpy as jnp
from jax.experimental import pallas as pl


def kernel(d_ij, r_ij, pair_indices, atom_index):
    raise NotImplementedError("write your pallas kernel here")


# speedup vs baseline: 11.8260x; 11.8260x over previous
import functools

import jax
import jax.numpy as jnp
import numpy as np
from jax.experimental import pallas as pl
from jax.experimental.pallas import tpu as pltpu

_NUM_SPECIES = 7
_N_RBF = 16
_R_MAX = 0.51
_R_MIN = 0.08
_A_MAX = 0.35
_A_MIN = 0.08
_ANG_DIV = 8
_ANG_SEC = 4
_RADIAL_ETA = 1970.0
_ANGULAR_ETA = 1250.0
_ZETA = 14.1
_NUM_PAIRS = _NUM_SPECIES * (_NUM_SPECIES + 1) // 2

_B = 16
_C = 128


def _aev_kernel(offs_ref, dmax_ref,
                sa_ref, spo_ref, close_ref, vx_ref, vy_ref, vz_ref,
                sd_ref, fca_ref, fcr_ref,
                rad_ref, ang_ref,
                *, total):
    g = pl.program_id(0)
    a0 = g * _B
    pos_start = offs_ref[a0]
    pos_end = offs_ref[a0 + _B]

    rad_ref[:, :] = jnp.zeros((_B * _NUM_SPECIES, _N_RBF), jnp.float32)
    ang_ref[:, :] = jnp.zeros((_B * _NUM_PAIRS, _ANG_SEC * _ANG_DIV), jnp.float32)

    ir = jax.lax.broadcasted_iota(jnp.int32, (_N_RBF, 1), 0).astype(jnp.float32)
    shifts_r = _R_MIN + (_R_MAX - _R_MIN) / _N_RBF * ir
    shfz_np = (np.arange(_ANG_SEC) + 0.5) * np.pi / _ANG_SEC
    cz_np = np.cos(shfz_np)
    sz_np = np.sin(shfz_np)
    iz = jax.lax.broadcasted_iota(jnp.int32, (_ANG_SEC, 1, 1), 0).astype(jnp.float32)

    def _sel4(vals):
        out = jnp.full((_ANG_SEC, 1, 1), float(vals[-1]), jnp.float32)
        for k in range(_ANG_SEC - 2, -1, -1):
            out = jnp.where(iz < (k + 0.5), float(vals[k]), out)
        return out

    cz = _sel4(cz_np)
    sz = _sel4(sz_np)
    ia = jax.lax.broadcasted_iota(jnp.int32, (1, _ANG_DIV, 1), 1).astype(jnp.float32)
    shfa = _A_MIN + (_A_MAX - _A_MIN) / _ANG_DIV * ia

    rb = pos_start // _C
    abase = rb * _C
    nchunks = (pos_end - abase + _C - 1) // _C

    rad_rows = jax.lax.broadcasted_iota(jnp.int32, (_B * _NUM_SPECIES, _C), 0)
    ang_rows = jax.lax.broadcasted_iota(jnp.int32, (_B * _NUM_PAIRS, _C), 0)

    def chunk_body(q, _):
        base = abase + q * _C
        sl = pl.ds((rb + q) * _C, _C)
        sa1 = sa_ref[0, sl].reshape(1, _C)
        spo1 = spo_ref[0, sl].reshape(1, _C)
        cl1 = close_ref[0, sl].reshape(1, _C)
        vx1 = vx_ref[0, sl].reshape(1, _C)
        vy1 = vy_ref[0, sl].reshape(1, _C)
        vz1 = vz_ref[0, sl].reshape(1, _C)
        sd1 = sd_ref[0, sl].reshape(1, _C)
        fca1 = fca_ref[0, sl].reshape(1, _C)

        local1 = sa1 - a0

        fcr1 = fcr_ref[0, sl].reshape(1, _C)
        rt = 0.25 * jnp.exp(-_RADIAL_ETA * (sd1 - shifts_r) ** 2) * fcr1
        rrow = local1 * _NUM_SPECIES + spo1
        oh_r = (rad_rows == rrow).astype(jnp.float32)
        rad_ref[:, :] += jax.lax.dot_general(
            oh_r, rt, (((1,), (1,)), ((), ())),
            preferred_element_type=jnp.float32)

        nd = jnp.minimum(dmax_ref[g], total - base)

        def delta_body(delta, _):
            w = delta // _C
            rem = delta - w * _C
            sl2 = pl.ds((rb + q + w) * _C, 2 * _C)
            rot = 2 * _C - rem

            def part(ref):
                dw = ref[0, sl2].reshape(1, 2 * _C)
                return pltpu.roll(dw, rot, 1)[:, :_C]

            sa2 = part(sa_ref)
            spo2 = part(spo_ref)
            cl2 = part(close_ref)
            vx2 = part(vx_ref)
            vy2 = part(vy_ref)
            vz2 = part(vz_ref)
            sd2 = part(sd_ref)
            fca2 = part(fca_ref)

            valid = (sa2 == sa1) & (cl1 > 0) & (cl2 > 0)

            dot = vx1 * vx2 + vy1 * vy2 + vz1 * vz2
            cosang = 0.95 * dot / (sd1 * sd2 + 1e-10)
            cosang_c = jnp.clip(cosang, -1.0, 1.0)
            sinang = jnp.sqrt(jnp.maximum(1.0 - cosang_c * cosang_c, 0.0))
            base1 = (1.0 + cosang_c * cz + sinang * sz) * 0.5
            f1 = jnp.exp(_ZETA * jnp.log(jnp.maximum(base1, 1e-30)))
            dmean = (sd1 + sd2) * 0.5
            f2 = jnp.exp(-_ANGULAR_ETA * (dmean - shfa) ** 2)
            at = 2.0 * f1 * f2 * (fca1 * fca2)
            at = at.reshape(_ANG_SEC * _ANG_DIV, _C)
            at = jnp.where(valid, at, 0.0)

            im = jnp.minimum(spo1, spo2)
            jm = jnp.maximum(spo1, spo2)
            trow = im * (2 * _NUM_SPECIES + 1 - im) // 2 + (jm - im)
            arow = local1 * _NUM_PAIRS + trow
            oh_a = ((ang_rows == arow) & valid).astype(jnp.float32)
            ang_ref[:, :] += jax.lax.dot_general(
                oh_a, at, (((1,), (1,)), ((), ())),
                preferred_element_type=jnp.float32)
            return 0

        jax.lax.fori_loop(1, nd + 1, delta_body, 0)
        return 0

    jax.lax.fori_loop(0, nchunks, chunk_body, 0)


def kernel(d_ij, r_ij, pair_indices, atom_index):
    N = atom_index.shape[0]
    P = pair_indices.shape[1]
    total = 2 * P
    d = d_ij.reshape(-1).astype(jnp.float32)

    ai = jnp.concatenate([pair_indices[0], pair_indices[1]]).astype(jnp.int32)
    order = jnp.argsort(ai).astype(jnp.int32)
    col = order % P
    sign = jnp.where(order < P, 1.0, -1.0).astype(jnp.float32)
    sa = ai[order]
    vec = r_ij[col] * sign[:, None]
    sd = d[col]
    other = jnp.where(order < P, pair_indices[1][col], pair_indices[0][col])
    spo = atom_index[other].astype(jnp.int32)
    close = (sd <= _A_MAX).astype(jnp.int32)
    fca = jnp.where(
        sd < _A_MAX, 0.5 * (jnp.cos(jnp.pi * sd / _A_MAX) + 1.0), 0.0
    ).astype(jnp.float32)
    fcr = jnp.where(
        sd < _R_MAX, 0.5 * (jnp.cos(jnp.pi * sd / _R_MAX) + 1.0), 0.0
    ).astype(jnp.float32)

    counts = jnp.bincount(ai, length=N).astype(jnp.int32)
    offsets = jnp.zeros((N + 1,), jnp.int32).at[1:].set(
        jnp.cumsum(counts).astype(jnp.int32))
    nb = N // _B
    dmaxg = counts.reshape(nb, _B).max(axis=1) - 1

    def pad_f(x, v):
        return jnp.concatenate(
            [x.astype(jnp.float32), jnp.full((2 * _C,), v, jnp.float32)]
        ).reshape(1, total + 2 * _C)

    def pad_i(x, v):
        return jnp.concatenate(
            [x.astype(jnp.int32), jnp.full((2 * _C,), v, jnp.int32)]
        ).reshape(1, total + 2 * _C)

    sa_p = pad_i(sa, N)
    spo_p = pad_i(spo, 0)
    close_p = pad_i(close, 0)
    vx_p = pad_f(vec[:, 0], 0.0)
    vy_p = pad_f(vec[:, 1], 0.0)
    vz_p = pad_f(vec[:, 2], 0.0)
    sd_p = pad_f(sd, 1e3)
    fca_p = pad_f(fca, 0.0)
    fcr_p = pad_f(fcr, 0.0)

    full_spec = pl.BlockSpec((1, total + 2 * _C), lambda g, *_: (0, 0))
    grid_spec = pltpu.PrefetchScalarGridSpec(
        num_scalar_prefetch=2,
        grid=(nb,),
        in_specs=[full_spec] * 9,
        out_specs=[
            pl.BlockSpec((_B * _NUM_SPECIES, _N_RBF), lambda g, *_: (g, 0)),
            pl.BlockSpec((_B * _NUM_PAIRS, _ANG_SEC * _ANG_DIV),
                         lambda g, *_: (g, 0)),
        ],
    )
    rad, ang = pl.pallas_call(
        functools.partial(_aev_kernel, total=total),
        grid_spec=grid_spec,
        out_shape=[
            jax.ShapeDtypeStruct((N * _NUM_SPECIES, _N_RBF), jnp.float32),
            jax.ShapeDtypeStruct((N * _NUM_PAIRS, _ANG_SEC * _ANG_DIV),
                                 jnp.float32),
        ],
    )(offsets, dmaxg, sa_p, spo_p, close_p, vx_p, vy_p, vz_p, sd_p,
      fca_p, fcr_p)
    return jnp.concatenate(
        [rad.reshape(N, _NUM_SPECIES * _N_RBF),
         ang.reshape(N, _NUM_PAIRS * _ANG_SEC * _ANG_DIV)], axis=-1)
